# baseline (speedup 1.0000x reference)
import functools

import jax
import jax.numpy as jnp
from jax import lax
from jax.experimental import pallas as pl
from jax.experimental.pallas import tpu as pltpu
from jax.experimental.pallas import tpu_sc as plsc

_VOCAB = 1000000
_EMB = 64
_PAD = 128
_BATCH = 16384
_HIST = 200
_B = _BATCH * _HIST
_NW = 32
_BPW = _B // _NW
_CHUNK = 200
_NCHUNK = _BPW // _CHUNK
_GBUF = 4


def _make_lookup():
    mesh = plsc.VectorSubcoreMesh(core_axis_name="c", subcore_axis_name="s")

    @functools.partial(
        pl.kernel,
        mesh=mesh,
        out_type=jax.ShapeDtypeStruct((_BATCH, _HIST, _PAD), jnp.float32),
        scratch_types=(
            [pltpu.VMEM((_CHUNK,), jnp.int32) for _ in range(_GBUF)]
            + [pltpu.VMEM((_CHUNK, _PAD), jnp.float32) for _ in range(_GBUF)]
            + [pltpu.SemaphoreType.DMA for _ in range(3 * _GBUF)]
        ),
        compiler_params=pltpu.CompilerParams(use_tc_tiling_on_sc=True),
    )
    def lookup(idx_hbm, table_hbm, out_hbm, *bufs):
        idx_v = bufs[:_GBUF]
        rows_v = bufs[_GBUF:2 * _GBUF]
        sem_i = bufs[2 * _GBUF:3 * _GBUF]
        sem_g = bufs[3 * _GBUF:4 * _GBUF]
        sem_s = bufs[4 * _GBUF:5 * _GBUF]
        wid = lax.axis_index("s") * 2 + lax.axis_index("c")
        base = wid * _BPW

        def idx_src(g):
            return idx_hbm.at[pl.ds(base + g * _CHUNK, _CHUNK)]

        def out_dst(g):
            return out_hbm.at[(base + g * _CHUNK) // _HIST]

        for s in range(_GBUF):
            pltpu.async_copy(idx_src(s), idx_v[s], sem_i[s])
        for s in range(2):
            pltpu.make_async_copy(idx_src(s), idx_v[s], sem_i[s]).wait()
            pltpu.async_copy(table_hbm.at[idx_v[s]], rows_v[s], sem_g[s])

        def body(g0, carry):
            for k in range(_GBUF):
                g = g0 * _GBUF + k
                pltpu.make_async_copy(table_hbm.at[idx_v[k]],
                                      rows_v[k], sem_g[k]).wait()
                pltpu.async_copy(rows_v[k], out_dst(g), sem_s[k])
                @pl.when(g + _GBUF < _NCHUNK)
                def _():
                    pltpu.async_copy(idx_src(g + _GBUF), idx_v[k], sem_i[k])
                kg = (k + 2) % _GBUF
                @pl.when(g + 2 < _NCHUNK)
                def _():
                    @pl.when(g >= 2)
                    def _():
                        pltpu.make_async_copy(rows_v[kg], out_dst(0),
                                              sem_s[kg]).wait()
                    pltpu.make_async_copy(idx_src(g + 2), idx_v[kg],
                                          sem_i[kg]).wait()
                    pltpu.async_copy(table_hbm.at[idx_v[kg]],
                                     rows_v[kg], sem_g[kg])
            return carry

        lax.fori_loop(0, _NCHUNK // _GBUF, body, 0)

        for k in range(_GBUF):
            pltpu.make_async_copy(rows_v[k], out_dst(0), sem_s[k]).wait()

    return lookup


_lookup = _make_lookup()


@jax.jit
def kernel(indices, table):
    table_p = jnp.pad(table, ((0, 0), (0, _PAD - _EMB)))
    out = _lookup(indices.reshape(_B), table_p)
    return out[:, :, :_EMB]
